# baseline (speedup 1.0000x reference)
import jax
import jax.numpy as jnp
from jax.experimental import pallas as pl
from jax.experimental.pallas import tpu as pltpu

_CB = 32
_K = 4


def _body(x_ref, mid_ref, o_ref, ibuf, obuf, isem, osem):
    bsz = x_ref.shape[0]
    c = x_ref.shape[1]
    hw = x_ref.shape[-1]
    n = c // _CB
    pos = jax.lax.broadcasted_iota(jnp.int32, (1, 1, hw), 2)
    mids = mid_ref[...].reshape(bsz, 1, 1)

    def in_copy(k):
        return pltpu.make_async_copy(
            x_ref.at[:, pl.ds(k * _CB, _CB), :], ibuf.at[k % _K], isem.at[k % _K])

    def out_copy(k):
        return pltpu.make_async_copy(
            obuf.at[k % _K], o_ref.at[:, pl.ds(k * _CB, _CB), :], osem.at[k % _K])

    for k in range(_K):
        in_copy(k).start(priority=k % 2)
    for k in range(n):
        in_copy(k).wait()
        if k >= _K:
            out_copy(k - _K).wait()
        obuf[k % _K] = jnp.where(pos == mids, 0.0, ibuf[k % _K])
        out_copy(k).start(priority=k % 2)
        if k + _K < n:
            in_copy(k + _K).start(priority=(k + _K) % 2)
    for k in range(n - _K, n):
        out_copy(k).wait()


def kernel(fea, mask_id):
    b, c, h, w = fea.shape
    hw = h * w
    x = fea.reshape(b, c, hw)
    out = pl.pallas_call(
        _body,
        grid=(1,),
        in_specs=[
            pl.BlockSpec(memory_space=pl.ANY),
            pl.BlockSpec(memory_space=pltpu.VMEM),
        ],
        out_specs=pl.BlockSpec(memory_space=pl.ANY),
        out_shape=jax.ShapeDtypeStruct((b, c, hw), jnp.float32),
        scratch_shapes=[
            pltpu.VMEM((_K, b, _CB, hw), jnp.float32),
            pltpu.VMEM((_K, b, _CB, hw), jnp.float32),
            pltpu.SemaphoreType.DMA((_K,)),
            pltpu.SemaphoreType.DMA((_K,)),
        ],
    )(x, mask_id[:, None])
    return out.reshape(b, c, h, w)
